# Initial kernel scaffold; baseline (speedup 1.0000x reference)
import jax
import jax.numpy as jnp
from jax.experimental import pallas as pl


def kernel(x, edge_index, batch, climber, W1, att_src1, att_dst1, b1, g1, be1, W2, att_src2, att_dst2, b2, g2, be2, Wc, bc, Wa, ba, Wb, bb):
    raise NotImplementedError("write your pallas kernel here")


# speedup vs baseline: 1.0989x; 1.0989x over previous
import functools
import jax
import jax.numpy as jnp
from jax.experimental import pallas as pl
from jax.experimental.pallas import tpu as pltpu

_N = 50000
_HEADS1 = 2
_HID = 32
_ROW_BLK = 1024


def _mm_att_body(x_ref, w_ref, asrc_ref, adst_ref, h_ref, av_src_ref, av_dst_ref):
    h = jnp.dot(x_ref[...], w_ref[...], preferred_element_type=jnp.float32)
    h_ref[...] = h
    heads = asrc_ref.shape[0]
    ch = asrc_ref.shape[1]
    hh = h.reshape(h.shape[0], heads, ch)
    av_src_ref[...] = (hh * asrc_ref[...][None]).sum(-1)
    av_dst_ref[...] = (hh * adst_ref[...][None]).sum(-1)


def _mm_att(x, w, a_src, a_dst):
    n, k = x.shape
    m = w.shape[1]
    heads = a_src.shape[0]
    grid = (pl.cdiv(n, _ROW_BLK),)
    return pl.pallas_call(
        _mm_att_body,
        grid=grid,
        in_specs=[
            pl.BlockSpec((_ROW_BLK, k), lambda i: (i, 0)),
            pl.BlockSpec((k, m), lambda i: (0, 0)),
            pl.BlockSpec((heads, m // heads), lambda i: (0, 0)),
            pl.BlockSpec((heads, m // heads), lambda i: (0, 0)),
        ],
        out_specs=[
            pl.BlockSpec((_ROW_BLK, m), lambda i: (i, 0)),
            pl.BlockSpec((_ROW_BLK, heads), lambda i: (i, 0)),
            pl.BlockSpec((_ROW_BLK, heads), lambda i: (i, 0)),
        ],
        out_shape=[
            jax.ShapeDtypeStruct((n, m), jnp.float32),
            jax.ShapeDtypeStruct((n, heads), jnp.float32),
            jax.ShapeDtypeStruct((n, heads), jnp.float32),
        ],
    )(x, w, a_src, a_dst)


def _final_body(h_ref, batch_ref, ce2_ref, wah_ref, ba_ref, wb_ref, bb_ref, out_ref):
    feat = h_ref[...]
    onehot = (batch_ref[...][:, None] == jax.lax.broadcasted_iota(jnp.int32, (1, 64), 1)).astype(jnp.float32)
    hid = jnp.dot(feat, wah_ref[...], preferred_element_type=jnp.float32)
    hid = hid + jnp.dot(onehot, ce2_ref[...], preferred_element_type=jnp.float32)
    hid = jax.nn.relu(hid + ba_ref[...][None])
    out_ref[...] = jnp.dot(hid, wb_ref[...], preferred_element_type=jnp.float32) + bb_ref[...][None]


def _final_stage(h2, batch, ce2, wa_h, ba, wb, bb):
    n = h2.shape[0]
    out_dim = wb.shape[1]
    grid = (pl.cdiv(n, _ROW_BLK),)
    return pl.pallas_call(
        _final_body,
        grid=grid,
        in_specs=[
            pl.BlockSpec((_ROW_BLK, h2.shape[1]), lambda i: (i, 0)),
            pl.BlockSpec((_ROW_BLK,), lambda i: (i,)),
            pl.BlockSpec((64, ce2.shape[1]), lambda i: (0, 0)),
            pl.BlockSpec(wa_h.shape, lambda i: (0, 0)),
            pl.BlockSpec(ba.shape, lambda i: (0,)),
            pl.BlockSpec(wb.shape, lambda i: (0, 0)),
            pl.BlockSpec(bb.shape, lambda i: (0,)),
        ],
        out_specs=pl.BlockSpec((_ROW_BLK, out_dim), lambda i: (i, 0)),
        out_shape=jax.ShapeDtypeStruct((n, out_dim), jnp.float32),
    )(h2, batch, ce2, wa_h, ba, wb, bb)


def _edge_softmax_scatter(h, asrc_n, adst_n, src, dst, heads, ch):
    n = h.shape[0]
    c = jnp.max(asrc_n, axis=0) + jnp.max(adst_n, axis=0)
    e = jax.nn.leaky_relu(asrc_n[src] + adst_n[dst], negative_slope=0.2)
    t = jnp.exp(e - c[None, :])
    denom = jax.ops.segment_sum(t, dst, num_segments=n)
    alpha = t / denom[dst]
    msg = h.reshape(-1, heads, ch)[src] * alpha[:, :, None]
    out = jax.ops.segment_sum(msg, dst, num_segments=n)
    return out.reshape(n, heads * ch)


def _bn_relu(x, bias, gamma, beta, eps=1e-5):
    x = x + bias[None]
    mean = x.mean(axis=0)
    var = (x * x).mean(axis=0) - mean * mean
    return jax.nn.relu(gamma * (x - mean) / jnp.sqrt(var + eps) + beta)


def kernel(x, edge_index, batch, climber, W1, att_src1, att_dst1, b1, g1, be1,
           W2, att_src2, att_dst2, b2, g2, be2, Wc, bc, Wa, ba, Wb, bb):
    n = x.shape[0]
    loop = jnp.arange(n, dtype=edge_index.dtype)
    src = jnp.concatenate([edge_index[0], loop])
    dst = jnp.concatenate([edge_index[1], loop])

    h1, av_src1, av_dst1 = _mm_att(x, W1, att_src1, att_dst1)
    o1 = _edge_softmax_scatter(h1, av_src1, av_dst1, src, dst, _HEADS1, _HID)
    a1 = _bn_relu(o1, b1, g1, be1)

    h2, av_src2, av_dst2 = _mm_att(a1, W2, att_src2, att_dst2)
    o2 = _edge_softmax_scatter(h2, av_src2, av_dst2, src, dst, 1, _HID)
    a2 = _bn_relu(o2, b2, g2, be2)

    ce = jax.nn.relu(climber @ Wc + bc[None])
    ce2 = ce @ Wa[_HID:]
    out = _final_stage(a2, batch.astype(jnp.int32), ce2, Wa[:_HID], ba, Wb, bb)
    return out


# speedup vs baseline: 54.3235x; 49.4362x over previous
import functools
import jax
import jax.numpy as jnp
from jax import lax
from jax.experimental import pallas as pl
from jax.experimental.pallas import tpu as pltpu
from jax.experimental.pallas import tpu_sc as plsc

_N = 50000
_NPAD = 50176
_STRIPE = _NPAD // 16
_E_TOTAL = 850000
_EPAD = 851968
_EROWS = _EPAD // 128
_ECHUNK = 1024
_TILE_EDGES = _EPAD // 16
_CHUNKS = _TILE_EDGES // _ECHUNK
_QROWS = _ECHUNK // 128
_BLK = 1024
_GRID = _NPAD // _BLK


def _enc_common(h, i, asrc, adst, tbl_ref, avs_ref, avd_ref, ms_ref, md_ref):
    nb = h.shape[0]
    split = tbl_ref.shape[0]
    ch = tbl_ref.shape[2]
    tbl_ref[...] = h.reshape(nb, split, ch).transpose(1, 0, 2)
    heads = avs_ref.shape[0]
    hch = h.shape[1] // heads
    hh = h.reshape(nb, heads, hch)
    avs = (hh * asrc[None]).sum(-1)
    avd = (hh * adst[None]).sum(-1)
    avs_ref[...] = avs.T
    avd_ref[...] = avd.T
    rows = i * _BLK + lax.broadcasted_iota(jnp.int32, (nb, 1), 0)
    valid = rows < _N
    ms = jnp.max(jnp.where(valid, avs, -3.4e38), axis=0)
    md = jnp.max(jnp.where(valid, avd, -3.4e38), axis=0)

    @pl.when(i == 0)
    def _():
        ms_ref[...] = jnp.full(ms_ref.shape, -3.4e38, jnp.float32)
        md_ref[...] = jnp.full(md_ref.shape, -3.4e38, jnp.float32)

    ms_ref[...] = jnp.maximum(ms_ref[...], ms[:, None])
    md_ref[...] = jnp.maximum(md_ref[...], md[:, None])


def _enc1_body(x_ref, w_ref, as_ref, ad_ref, tbl_ref, avs_ref, avd_ref, ms_ref, md_ref):
    i = pl.program_id(0)
    h = jnp.dot(x_ref[...], w_ref[...], preferred_element_type=jnp.float32)
    _enc_common(h, i, as_ref[...], ad_ref[...], tbl_ref, avs_ref, avd_ref, ms_ref, md_ref)


def _enc2_body(sc_ref, scale_ref, shift_ref, w_ref, as_ref, ad_ref,
               tbl_ref, avs_ref, avd_ref, ms_ref, md_ref):
    i = pl.program_id(0)
    t = sc_ref[...]
    chin2 = t.shape[0] * t.shape[2]
    act = t.transpose(1, 0, 2).reshape(_BLK, chin2)
    act = jax.nn.relu(act * scale_ref[...][0][None] + shift_ref[...][0][None])
    h = jnp.dot(act, w_ref[...], preferred_element_type=jnp.float32)
    _enc_common(h, i, as_ref[...], ad_ref[...], tbl_ref, avs_ref, avd_ref, ms_ref, md_ref)


def _enc1(x, w, a_src, a_dst, split, ch):
    heads = a_src.shape[0]
    m = w.shape[1]
    return pl.pallas_call(
        _enc1_body,
        grid=(_GRID,),
        in_specs=[
            pl.BlockSpec((_BLK, x.shape[1]), lambda i: (i, 0)),
            pl.BlockSpec(w.shape, lambda i: (0, 0)),
            pl.BlockSpec(a_src.shape, lambda i: (0, 0)),
            pl.BlockSpec(a_dst.shape, lambda i: (0, 0)),
        ],
        out_specs=[
            pl.BlockSpec((split, _BLK, ch), lambda i: (0, i, 0)),
            pl.BlockSpec((heads, _BLK), lambda i: (0, i)),
            pl.BlockSpec((heads, _BLK), lambda i: (0, i)),
            pl.BlockSpec((heads, 16), lambda i: (0, 0)),
            pl.BlockSpec((heads, 16), lambda i: (0, 0)),
        ],
        out_shape=[
            jax.ShapeDtypeStruct((split, _NPAD, ch), jnp.float32),
            jax.ShapeDtypeStruct((heads, _NPAD), jnp.float32),
            jax.ShapeDtypeStruct((heads, _NPAD), jnp.float32),
            jax.ShapeDtypeStruct((heads, 16), jnp.float32),
            jax.ShapeDtypeStruct((heads, 16), jnp.float32),
        ],
    )(x, w, a_src, a_dst)


def _enc2(sc_in, scale, shift, w, a_src, a_dst, split, ch):
    heads = a_src.shape[0]
    chin = sc_in.shape[2]
    return pl.pallas_call(
        _enc2_body,
        grid=(_GRID,),
        in_specs=[
            pl.BlockSpec((2, _BLK, chin), lambda i: (0, i, 0)),
            pl.BlockSpec(scale.shape, lambda i: (0, 0)),
            pl.BlockSpec(shift.shape, lambda i: (0, 0)),
            pl.BlockSpec(w.shape, lambda i: (0, 0)),
            pl.BlockSpec(a_src.shape, lambda i: (0, 0)),
            pl.BlockSpec(a_dst.shape, lambda i: (0, 0)),
        ],
        out_specs=[
            pl.BlockSpec((split, _BLK, ch), lambda i: (0, i, 0)),
            pl.BlockSpec((heads, _BLK), lambda i: (0, i)),
            pl.BlockSpec((heads, _BLK), lambda i: (0, i)),
            pl.BlockSpec((heads, 16), lambda i: (0, 0)),
            pl.BlockSpec((heads, 16), lambda i: (0, 0)),
        ],
        out_shape=[
            jax.ShapeDtypeStruct((split, _NPAD, ch), jnp.float32),
            jax.ShapeDtypeStruct((heads, _NPAD), jnp.float32),
            jax.ShapeDtypeStruct((heads, _NPAD), jnp.float32),
            jax.ShapeDtypeStruct((heads, 16), jnp.float32),
            jax.ShapeDtypeStruct((heads, 16), jnp.float32),
        ],
    )(sc_in, scale, shift, w, a_src, a_dst)


def _red_body(sc_ref, sum_ref, sq_ref):
    i = pl.program_id(0)
    t = sc_ref[...]
    m = t.transpose(1, 0, 2).reshape(_BLK, t.shape[0] * t.shape[2])

    @pl.when(i == 0)
    def _():
        sum_ref[...] = jnp.zeros(sum_ref.shape, jnp.float32)
        sq_ref[...] = jnp.zeros(sq_ref.shape, jnp.float32)

    sum_ref[...] += m.sum(0)[None]
    sq_ref[...] += (m * m).sum(0)[None]


def _red(sc_in):
    chin = sc_in.shape[2]
    m = 2 * chin
    return pl.pallas_call(
        _red_body,
        grid=(_GRID,),
        in_specs=[pl.BlockSpec((2, _BLK, chin), lambda i: (0, i, 0))],
        out_specs=[
            pl.BlockSpec((1, m), lambda i: (0, 0)),
            pl.BlockSpec((1, m), lambda i: (0, 0)),
        ],
        out_shape=[
            jax.ShapeDtypeStruct((1, m), jnp.float32),
            jax.ShapeDtypeStruct((1, m), jnp.float32),
        ],
    )(sc_in)


def _fin_body(sc_ref, scale_ref, shift_ref, batch_ref, cl_ref, wc_ref, bc_ref,
              wa_ref, ba_ref, wb_ref, bb_ref, out_ref):
    t = sc_ref[...]
    act = t.transpose(1, 0, 2).reshape(_BLK, 32)
    act = jax.nn.relu(act * scale_ref[...][0][None] + shift_ref[...][0][None])
    ce = jax.nn.relu(jnp.dot(cl_ref[...], wc_ref[...], preferred_element_type=jnp.float32)
                     + bc_ref[...][0][None])
    ce2 = jnp.dot(ce, wa_ref[...][32:, :], preferred_element_type=jnp.float32)
    oh = (batch_ref[...] == lax.broadcasted_iota(jnp.int32, (1, 64), 1)).astype(jnp.float32)
    hid = jnp.dot(act, wa_ref[...][:32, :], preferred_element_type=jnp.float32)
    hid = hid + jnp.dot(oh, ce2, preferred_element_type=jnp.float32)
    hid = jax.nn.relu(hid + ba_ref[...][0][None])
    out_ref[...] = jnp.dot(hid, wb_ref[...], preferred_element_type=jnp.float32) + bb_ref[...][0][None]


def _fin(sc_in, scale, shift, batch_p, climber, wc, bc, wa, ba, wb, bb):
    return pl.pallas_call(
        _fin_body,
        grid=(_GRID,),
        in_specs=[
            pl.BlockSpec((2, _BLK, 16), lambda i: (0, i, 0)),
            pl.BlockSpec(scale.shape, lambda i: (0, 0)),
            pl.BlockSpec(shift.shape, lambda i: (0, 0)),
            pl.BlockSpec((_BLK, 1), lambda i: (i, 0)),
            pl.BlockSpec(climber.shape, lambda i: (0, 0)),
            pl.BlockSpec(wc.shape, lambda i: (0, 0)),
            pl.BlockSpec(bc.shape, lambda i: (0, 0)),
            pl.BlockSpec(wa.shape, lambda i: (0, 0)),
            pl.BlockSpec(ba.shape, lambda i: (0, 0)),
            pl.BlockSpec(wb.shape, lambda i: (0, 0)),
            pl.BlockSpec(bb.shape, lambda i: (0, 0)),
        ],
        out_specs=pl.BlockSpec((_BLK, 4), lambda i: (i, 0)),
        out_shape=jax.ShapeDtypeStruct((_NPAD, 4), jnp.float32),
    )(sc_in, scale, shift, batch_p, climber, wc, bc, wa, ba, wb, bb)


def _make_edge_kernel(heads, ch):
    mesh = plsc.VectorSubcoreMesh(core_axis_name="c", subcore_axis_name="s")
    out_type = [
        jax.ShapeDtypeStruct((2, _NPAD, ch), jnp.float32),
        jax.ShapeDtypeStruct((2, _EROWS, 128), jnp.float32),
    ]
    scratch = [
        pltpu.VMEM_SHARED((_NPAD,), jnp.float32),
        pltpu.VMEM_SHARED((_NPAD,), jnp.float32),
        pltpu.VMEM_SHARED((_NPAD,), jnp.float32),
        pltpu.VMEM_SHARED((_NPAD, ch), jnp.float32),
        pltpu.VMEM((_QROWS, 128), jnp.int32),
        pltpu.VMEM((_QROWS, 128), jnp.int32),
        pltpu.VMEM((_QROWS, 128), jnp.float32),
        pltpu.VMEM((_QROWS, 128), jnp.float32),
        pltpu.VMEM((_QROWS, 128), jnp.float32),
        pltpu.VMEM((_QROWS, 128), jnp.float32),
        pltpu.VMEM((_QROWS, 128), jnp.float32),
        pltpu.VMEM((128, ch), jnp.float32),
        pltpu.VMEM((_STRIPE,), jnp.float32),
        pltpu.VMEM((64, ch), jnp.float32),
        pltpu.VMEM((16,), jnp.float32),
        pltpu.VMEM((16,), jnp.float32),
    ]

    @functools.partial(pl.kernel, out_type=out_type, mesh=mesh, scratch_types=scratch,
                       compiler_params=pltpu.CompilerParams(use_tc_tiling_on_sc=False))
    def edge_kernel(tbl_hbm, src_hbm, src2_hbm, dst_hbm, asrc_hbm, adst_hbm, ms_hbm, md_hbm,
                    out_hbm, t_hbm,
                    sh_asrc, sh_adst, sh_den, sh_acc,
                    srcv, dstv, av, bv, tv, gv, wv, rows, dv, zb2, c16a, c16b):
        core = lax.axis_index("c")
        sub = lax.axis_index("s")
        hidx = core if heads == 2 else 0
        sbase = pl.multiple_of(sub * _STRIPE, 8)
        aoff = pl.multiple_of(hidx * _NPAD + sub * _STRIPE, 8)
        moff = pl.multiple_of(hidx * 16, 8)

        pltpu.sync_copy(ms_hbm.at[pl.ds(moff, 16)], c16a)
        pltpu.sync_copy(md_hbm.at[pl.ds(moff, 16)], c16b)
        pltpu.sync_copy(asrc_hbm.at[pl.ds(aoff, _STRIPE)], dv)
        pltpu.sync_copy(dv, sh_asrc.at[pl.ds(sbase, _STRIPE)])
        pltpu.sync_copy(adst_hbm.at[pl.ds(aoff, _STRIPE)], dv)
        pltpu.sync_copy(dv, sh_adst.at[pl.ds(sbase, _STRIPE)])

        def zfill(j, _):
            dv[pl.ds(j * 16, 16)] = jnp.zeros((16,), jnp.float32)
            return 0
        lax.fori_loop(0, _STRIPE // 16, zfill, 0)

        def zfill2(r, _):
            for j in range(ch // 16):
                zb2[r, pl.ds(j * 16, 16)] = jnp.zeros((16,), jnp.float32)
            return 0
        lax.fori_loop(0, 64, zfill2, 0)

        pltpu.sync_copy(dv, sh_den.at[pl.ds(sbase, _STRIPE)])

        def zacc(k, _):
            pltpu.sync_copy(zb2, sh_acc.at[pl.ds(sbase + k * 64, 64)])
            return 0
        lax.fori_loop(0, _STRIPE // 64, zacc, 0)

        plsc.subcore_barrier()

        cvec = c16a[...] + c16b[...]

        def p1(i, _):
            rbase = pl.multiple_of(sub * (_TILE_EDGES // 128) + i * _QROWS, 8)
            pltpu.sync_copy(src_hbm.at[pl.ds(rbase, _QROWS)], srcv)
            pltpu.sync_copy(dst_hbm.at[pl.ds(rbase, _QROWS)], dstv)
            for q in range(_QROWS):
                pltpu.sync_copy(sh_asrc.at[srcv.at[q]], av.at[q])
                pltpu.sync_copy(sh_adst.at[dstv.at[q]], bv.at[q])
            ebase = rbase * 128
            for q in range(_QROWS):
                for j in range(8):
                    sl = pl.ds(j * 16, 16)
                    e = av[q, sl] + bv[q, sl]
                    e = jnp.maximum(e, 0.2 * e)
                    t = jnp.exp(e - cvec)
                    eid = ebase + q * 128 + j * 16 + lax.broadcasted_iota(jnp.int32, (16,), 0)
                    tv[q, sl] = jnp.where(eid < _E_TOTAL, t, 0.0)
            for q in range(_QROWS):
                pltpu.sync_copy(tv.at[q], sh_den.at[dstv.at[q]], add=True)
            pltpu.sync_copy(tv, t_hbm.at[core, pl.ds(rbase, _QROWS)])
            return 0
        lax.fori_loop(0, _CHUNKS, p1, 0)

        plsc.subcore_barrier()

        pltpu.sync_copy(sh_den.at[pl.ds(sbase, _STRIPE)], dv)

        def inv(j, _):
            sl = pl.ds(j * 16, 16)
            dv[sl] = 1.0 / (dv[sl] + 1e-16)
            return 0
        lax.fori_loop(0, _STRIPE // 16, inv, 0)
        pltpu.sync_copy(dv, sh_den.at[pl.ds(sbase, _STRIPE)])

        plsc.subcore_barrier()

        def p2(i, _):
            rbase = pl.multiple_of(sub * (_TILE_EDGES // 128) + i * _QROWS, 8)
            pltpu.sync_copy(t_hbm.at[core, pl.ds(rbase, _QROWS)], tv)
            pltpu.sync_copy(dst_hbm.at[pl.ds(rbase, _QROWS)], dstv)
            pltpu.sync_copy(src2_hbm.at[core, pl.ds(rbase, _QROWS)], srcv)
            for q in range(_QROWS):
                pltpu.sync_copy(sh_den.at[dstv.at[q]], gv.at[q])
            for q in range(_QROWS):
                for j in range(8):
                    sl = pl.ds(j * 16, 16)
                    wv[q, sl] = tv[q, sl] * gv[q, sl]
            for q in range(_QROWS):
                pltpu.sync_copy(tbl_hbm.at[srcv.at[q]], rows)

                def scale_rows(rg, _):
                    w16 = wv[q, pl.ds(rg * 16, 16)]
                    for k in range(16):
                        r = rg * 16 + k
                        wvec = jnp.broadcast_to(w16[k], (16,))
                        for j in range(ch // 16):
                            sl = pl.ds(j * 16, 16)
                            rows[r, sl] = rows[r, sl] * wvec
                    return 0
                lax.fori_loop(0, 8, scale_rows, 0)
                pltpu.sync_copy(rows, sh_acc.at[dstv.at[q]], add=True)
            return 0
        lax.fori_loop(0, _CHUNKS, p2, 0)

        plsc.subcore_barrier()

        def wb(k, _):
            off = pl.multiple_of(sbase + k * 64, 8)
            pltpu.sync_copy(sh_acc.at[pl.ds(off, 64)], zb2)
            pltpu.sync_copy(zb2, out_hbm.at[core, pl.ds(off, 64)])
            return 0
        lax.fori_loop(0, _STRIPE // 64, wb, 0)

    return edge_kernel


_make_edge_kernel = functools.lru_cache(maxsize=None)(_make_edge_kernel)


def _bn_coeffs(sumv, sqv, gamma, beta):
    mean = sumv[0] / _N
    var = sqv[0] / _N - mean * mean
    scale = gamma * lax.rsqrt(var + 1e-5)
    shift = beta - mean * scale
    return scale[None], shift[None]


def kernel(x, edge_index, batch, climber, W1, att_src1, att_dst1, b1, g1, be1,
           W2, att_src2, att_dst2, b2, g2, be2, Wc, bc, Wa, ba, Wb, bb):
    n = x.shape[0]
    loop = jnp.arange(n, dtype=jnp.int32)
    src = jnp.concatenate([edge_index[0].astype(jnp.int32), loop])
    dst = jnp.concatenate([edge_index[1].astype(jnp.int32), loop])
    srcp = jnp.pad(src, (0, _EPAD - _E_TOTAL))
    dstp = jnp.pad(dst, (0, _EPAD - _E_TOTAL))
    src_r = srcp.reshape(_EROWS, 128)
    dst_r = dstp.reshape(_EROWS, 128)
    src2_r = jnp.stack([srcp, srcp + _NPAD]).reshape(2, _EROWS, 128)
    batch_p = jnp.pad(batch.astype(jnp.int32), (0, _NPAD - n)).reshape(_NPAD, 1)

    tbl1, avs1, avd1, ms1, md1 = _enc1(x, W1, att_src1, att_dst1, 2, 32)
    out1, _ = _make_edge_kernel(2, 32)(tbl1.reshape(2 * _NPAD, 32), src_r, src2_r, dst_r,
                                       avs1.reshape(-1), avd1.reshape(-1),
                                       ms1.reshape(-1), md1.reshape(-1))
    sum1, sq1 = _red(out1)
    scale1, shift1 = _bn_coeffs(sum1, sq1, g1, be1)

    tbl2, avs2, avd2, ms2, md2 = _enc2(out1, scale1, shift1, W2, att_src2, att_dst2, 2, 16)
    out2, _ = _make_edge_kernel(1, 16)(tbl2.reshape(2 * _NPAD, 16), src_r, src2_r, dst_r,
                                       avs2.reshape(-1), avd2.reshape(-1),
                                       ms2.reshape(-1), md2.reshape(-1))
    sum2, sq2 = _red(out2)
    scale2, shift2 = _bn_coeffs(sum2, sq2, g2, be2)

    y = _fin(out2, scale2, shift2, batch_p, climber, Wc, bc.reshape(1, -1),
             Wa, ba.reshape(1, -1), Wb, bb.reshape(1, -1))
    return y[:n]


# speedup vs baseline: 76.9342x; 1.4162x over previous
import functools
import jax
import jax.numpy as jnp
from jax import lax
from jax.experimental import pallas as pl
from jax.experimental.pallas import tpu as pltpu
from jax.experimental.pallas import tpu_sc as plsc

_N = 50000
_NPAD = 50176
_STRIPE = _NPAD // 16
_E_TOTAL = 850000
_EPAD = 851968
_EROWS = _EPAD // 128
_ECHUNK = 1024
_TILE_EDGES = _EPAD // 16
_CHUNKS = _TILE_EDGES // _ECHUNK
_QROWS = _ECHUNK // 128
_BLK = 1024
_GRID = _NPAD // _BLK


def _enc_common(h, i, asrc, adst, tbl_ref, avs_ref, avd_ref, ms_ref, md_ref):
    nb = h.shape[0]
    split = tbl_ref.shape[0]
    ch = tbl_ref.shape[2]
    tbl_ref[...] = h.reshape(nb, split, ch).transpose(1, 0, 2)
    heads = avs_ref.shape[0]
    hch = h.shape[1] // heads
    hh = h.reshape(nb, heads, hch)
    avs = (hh * asrc[None]).sum(-1)
    avd = (hh * adst[None]).sum(-1)
    avs_ref[...] = avs.T
    avd_ref[...] = avd.T
    rows = i * _BLK + lax.broadcasted_iota(jnp.int32, (nb, 1), 0)
    valid = rows < _N
    ms = jnp.max(jnp.where(valid, avs, -3.4e38), axis=0)
    md = jnp.max(jnp.where(valid, avd, -3.4e38), axis=0)

    @pl.when(i == 0)
    def _():
        ms_ref[...] = jnp.full(ms_ref.shape, -3.4e38, jnp.float32)
        md_ref[...] = jnp.full(md_ref.shape, -3.4e38, jnp.float32)

    ms_ref[...] = jnp.maximum(ms_ref[...], ms[:, None])
    md_ref[...] = jnp.maximum(md_ref[...], md[:, None])


def _enc1_body(x_ref, w_ref, as_ref, ad_ref, tbl_ref, avs_ref, avd_ref, ms_ref, md_ref):
    i = pl.program_id(0)
    h = jnp.dot(x_ref[...], w_ref[...], preferred_element_type=jnp.float32)
    _enc_common(h, i, as_ref[...], ad_ref[...], tbl_ref, avs_ref, avd_ref, ms_ref, md_ref)


def _enc2_body(sc_ref, scale_ref, shift_ref, w_ref, as_ref, ad_ref,
               tbl_ref, avs_ref, avd_ref, ms_ref, md_ref):
    i = pl.program_id(0)
    t = sc_ref[...]
    chin2 = t.shape[0] * t.shape[2]
    act = t.transpose(1, 0, 2).reshape(_BLK, chin2)
    act = jax.nn.relu(act * scale_ref[...][0][None] + shift_ref[...][0][None])
    h = jnp.dot(act, w_ref[...], preferred_element_type=jnp.float32)
    _enc_common(h, i, as_ref[...], ad_ref[...], tbl_ref, avs_ref, avd_ref, ms_ref, md_ref)


def _enc1(x, w, a_src, a_dst, split, ch):
    heads = a_src.shape[0]
    m = w.shape[1]
    return pl.pallas_call(
        _enc1_body,
        grid=(_GRID,),
        in_specs=[
            pl.BlockSpec((_BLK, x.shape[1]), lambda i: (i, 0)),
            pl.BlockSpec(w.shape, lambda i: (0, 0)),
            pl.BlockSpec(a_src.shape, lambda i: (0, 0)),
            pl.BlockSpec(a_dst.shape, lambda i: (0, 0)),
        ],
        out_specs=[
            pl.BlockSpec((split, _BLK, ch), lambda i: (0, i, 0)),
            pl.BlockSpec((heads, _BLK), lambda i: (0, i)),
            pl.BlockSpec((heads, _BLK), lambda i: (0, i)),
            pl.BlockSpec((heads, 16), lambda i: (0, 0)),
            pl.BlockSpec((heads, 16), lambda i: (0, 0)),
        ],
        out_shape=[
            jax.ShapeDtypeStruct((split, _NPAD, ch), jnp.float32),
            jax.ShapeDtypeStruct((heads, _NPAD), jnp.float32),
            jax.ShapeDtypeStruct((heads, _NPAD), jnp.float32),
            jax.ShapeDtypeStruct((heads, 16), jnp.float32),
            jax.ShapeDtypeStruct((heads, 16), jnp.float32),
        ],
    )(x, w, a_src, a_dst)


def _enc2(sc_in, scale, shift, w, a_src, a_dst, split, ch):
    heads = a_src.shape[0]
    chin = sc_in.shape[2]
    return pl.pallas_call(
        _enc2_body,
        grid=(_GRID,),
        in_specs=[
            pl.BlockSpec((2, _BLK, chin), lambda i: (0, i, 0)),
            pl.BlockSpec(scale.shape, lambda i: (0, 0)),
            pl.BlockSpec(shift.shape, lambda i: (0, 0)),
            pl.BlockSpec(w.shape, lambda i: (0, 0)),
            pl.BlockSpec(a_src.shape, lambda i: (0, 0)),
            pl.BlockSpec(a_dst.shape, lambda i: (0, 0)),
        ],
        out_specs=[
            pl.BlockSpec((split, _BLK, ch), lambda i: (0, i, 0)),
            pl.BlockSpec((heads, _BLK), lambda i: (0, i)),
            pl.BlockSpec((heads, _BLK), lambda i: (0, i)),
            pl.BlockSpec((heads, 16), lambda i: (0, 0)),
            pl.BlockSpec((heads, 16), lambda i: (0, 0)),
        ],
        out_shape=[
            jax.ShapeDtypeStruct((split, _NPAD, ch), jnp.float32),
            jax.ShapeDtypeStruct((heads, _NPAD), jnp.float32),
            jax.ShapeDtypeStruct((heads, _NPAD), jnp.float32),
            jax.ShapeDtypeStruct((heads, 16), jnp.float32),
            jax.ShapeDtypeStruct((heads, 16), jnp.float32),
        ],
    )(sc_in, scale, shift, w, a_src, a_dst)


def _red_body(sc_ref, sum_ref, sq_ref):
    i = pl.program_id(0)
    t = sc_ref[...]
    m = t.transpose(1, 0, 2).reshape(_BLK, t.shape[0] * t.shape[2])

    @pl.when(i == 0)
    def _():
        sum_ref[...] = jnp.zeros(sum_ref.shape, jnp.float32)
        sq_ref[...] = jnp.zeros(sq_ref.shape, jnp.float32)

    sum_ref[...] += m.sum(0)[None]
    sq_ref[...] += (m * m).sum(0)[None]


def _red(sc_in):
    chin = sc_in.shape[2]
    m = 2 * chin
    return pl.pallas_call(
        _red_body,
        grid=(_GRID,),
        in_specs=[pl.BlockSpec((2, _BLK, chin), lambda i: (0, i, 0))],
        out_specs=[
            pl.BlockSpec((1, m), lambda i: (0, 0)),
            pl.BlockSpec((1, m), lambda i: (0, 0)),
        ],
        out_shape=[
            jax.ShapeDtypeStruct((1, m), jnp.float32),
            jax.ShapeDtypeStruct((1, m), jnp.float32),
        ],
    )(sc_in)


def _fin_body(sc_ref, scale_ref, shift_ref, batch_ref, cl_ref, wc_ref, bc_ref,
              wa_ref, ba_ref, wb_ref, bb_ref, out_ref):
    t = sc_ref[...]
    act = t.transpose(1, 0, 2).reshape(_BLK, 32)
    act = jax.nn.relu(act * scale_ref[...][0][None] + shift_ref[...][0][None])
    ce = jax.nn.relu(jnp.dot(cl_ref[...], wc_ref[...], preferred_element_type=jnp.float32)
                     + bc_ref[...][0][None])
    ce2 = jnp.dot(ce, wa_ref[...][32:, :], preferred_element_type=jnp.float32)
    oh = (batch_ref[...] == lax.broadcasted_iota(jnp.int32, (1, 64), 1)).astype(jnp.float32)
    hid = jnp.dot(act, wa_ref[...][:32, :], preferred_element_type=jnp.float32)
    hid = hid + jnp.dot(oh, ce2, preferred_element_type=jnp.float32)
    hid = jax.nn.relu(hid + ba_ref[...][0][None])
    out_ref[...] = jnp.dot(hid, wb_ref[...], preferred_element_type=jnp.float32) + bb_ref[...][0][None]


def _fin(sc_in, scale, shift, batch_p, climber, wc, bc, wa, ba, wb, bb):
    return pl.pallas_call(
        _fin_body,
        grid=(_GRID,),
        in_specs=[
            pl.BlockSpec((2, _BLK, 16), lambda i: (0, i, 0)),
            pl.BlockSpec(scale.shape, lambda i: (0, 0)),
            pl.BlockSpec(shift.shape, lambda i: (0, 0)),
            pl.BlockSpec((_BLK, 1), lambda i: (i, 0)),
            pl.BlockSpec(climber.shape, lambda i: (0, 0)),
            pl.BlockSpec(wc.shape, lambda i: (0, 0)),
            pl.BlockSpec(bc.shape, lambda i: (0, 0)),
            pl.BlockSpec(wa.shape, lambda i: (0, 0)),
            pl.BlockSpec(ba.shape, lambda i: (0, 0)),
            pl.BlockSpec(wb.shape, lambda i: (0, 0)),
            pl.BlockSpec(bb.shape, lambda i: (0, 0)),
        ],
        out_specs=pl.BlockSpec((_BLK, 4), lambda i: (i, 0)),
        out_shape=jax.ShapeDtypeStruct((_NPAD, 4), jnp.float32),
    )(sc_in, scale, shift, batch_p, climber, wc, bc, wa, ba, wb, bb)


def _make_edge_kernel(heads, ch):
    nb = 2 if ch == 32 else 8
    mesh = plsc.VectorSubcoreMesh(core_axis_name="c", subcore_axis_name="s")
    out_type = [
        jax.ShapeDtypeStruct((2, _NPAD, ch), jnp.float32),
        jax.ShapeDtypeStruct((2, _EROWS, 128), jnp.float32),
    ]
    scratch = [
        pltpu.VMEM_SHARED((_NPAD,), jnp.float32),
        pltpu.VMEM_SHARED((_NPAD,), jnp.float32),
        pltpu.VMEM_SHARED((_NPAD,), jnp.float32),
        pltpu.VMEM_SHARED((_NPAD, ch), jnp.float32),
        pltpu.VMEM((_QROWS, 128), jnp.int32),
        pltpu.VMEM((_QROWS, 128), jnp.int32),
        pltpu.VMEM((_QROWS, 128), jnp.float32),
        pltpu.VMEM((_QROWS, 128), jnp.float32),
        pltpu.VMEM((_QROWS, 128), jnp.float32),
        pltpu.VMEM((_QROWS, 128), jnp.float32),
        pltpu.VMEM((_QROWS, 128), jnp.float32),
        pltpu.VMEM((nb, 128, ch), jnp.float32),
        pltpu.VMEM((_STRIPE,), jnp.float32),
        pltpu.VMEM((64, ch), jnp.float32),
        pltpu.VMEM((16,), jnp.float32),
        pltpu.VMEM((16,), jnp.float32),
        pltpu.SemaphoreType.DMA,
        pltpu.SemaphoreType.DMA,
        pltpu.SemaphoreType.DMA,
        pltpu.SemaphoreType.DMA,
        pltpu.SemaphoreType.DMA,
        pltpu.SemaphoreType.DMA,
        pltpu.SemaphoreType.DMA,
        pltpu.SemaphoreType.DMA,
    ]

    @functools.partial(pl.kernel, out_type=out_type, mesh=mesh, scratch_types=scratch,
                       compiler_params=pltpu.CompilerParams(use_tc_tiling_on_sc=False))
    def edge_kernel(tbl_hbm, src_hbm, src2_hbm, dst_hbm, asrc_hbm, adst_hbm, ms_hbm, md_hbm,
                    out_hbm, t_hbm,
                    sh_asrc, sh_adst, sh_den, sh_acc,
                    srcv, dstv, av, bv, tv, gv, wv, rows, dv, zb2, c16a, c16b,
                    s_l, s_t, s_g, s_w, s_r0, s_r1, s_s0, s_s1):
        core = lax.axis_index("c")
        sub = lax.axis_index("s")
        hidx = core if heads == 2 else 0
        sbase = pl.multiple_of(sub * _STRIPE, 8)
        aoff = pl.multiple_of(hidx * _NPAD + sub * _STRIPE, 8)
        moff = pl.multiple_of(hidx * 16, 8)

        pltpu.sync_copy(ms_hbm.at[pl.ds(moff, 16)], c16a)
        pltpu.sync_copy(md_hbm.at[pl.ds(moff, 16)], c16b)
        pltpu.sync_copy(asrc_hbm.at[pl.ds(aoff, _STRIPE)], dv)
        pltpu.sync_copy(dv, sh_asrc.at[pl.ds(sbase, _STRIPE)])
        pltpu.sync_copy(adst_hbm.at[pl.ds(aoff, _STRIPE)], dv)
        pltpu.sync_copy(dv, sh_adst.at[pl.ds(sbase, _STRIPE)])

        def zfill(j, _):
            dv[pl.ds(j * 16, 16)] = jnp.zeros((16,), jnp.float32)
            return 0
        lax.fori_loop(0, _STRIPE // 16, zfill, 0)

        def zfill2(r, _):
            for j in range(ch // 16):
                zb2[r, pl.ds(j * 16, 16)] = jnp.zeros((16,), jnp.float32)
            return 0
        lax.fori_loop(0, 64, zfill2, 0)

        pltpu.sync_copy(dv, sh_den.at[pl.ds(sbase, _STRIPE)])

        def zacc(k, _):
            pltpu.sync_copy(zb2, sh_acc.at[pl.ds(sbase + k * 64, 64)])
            return 0
        lax.fori_loop(0, _STRIPE // 64, zacc, 0)

        plsc.subcore_barrier()

        cvec = c16a[...] + c16b[...]

        def p1(i, _):
            rbase = pl.multiple_of(sub * (_TILE_EDGES // 128) + i * _QROWS, 8)
            la = pltpu.async_copy(src_hbm.at[pl.ds(rbase, _QROWS)], srcv, s_l)
            lb = pltpu.async_copy(dst_hbm.at[pl.ds(rbase, _QROWS)], dstv, s_l)
            la.wait()
            lb.wait()
            gd = []
            for q in range(_QROWS):
                gd.append(pltpu.async_copy(sh_asrc.at[srcv.at[q]], av.at[q], s_g))
                gd.append(pltpu.async_copy(sh_adst.at[dstv.at[q]], bv.at[q], s_g))
            for d in gd:
                d.wait()

            ebase = rbase * 128
            for q in range(_QROWS):
                for j in range(8):
                    sl = pl.ds(j * 16, 16)
                    e = av[q, sl] + bv[q, sl]
                    e = jnp.maximum(e, 0.2 * e)
                    t = jnp.exp(e - cvec)
                    eid = ebase + q * 128 + j * 16 + lax.broadcasted_iota(jnp.int32, (16,), 0)
                    tv[q, sl] = jnp.where(eid < _E_TOTAL, t, 0.0)
            wd = [pltpu.async_copy(tv.at[q], sh_den.at[dstv.at[q]], s_w, add=True)
                  for q in range(_QROWS)]
            ts = pltpu.async_copy(tv, t_hbm.at[core, pl.ds(rbase, _QROWS)], s_t)
            for d in wd:
                d.wait()
            ts.wait()
            return 0
        lax.fori_loop(0, _CHUNKS, p1, 0)

        plsc.subcore_barrier()

        pltpu.sync_copy(sh_den.at[pl.ds(sbase, _STRIPE)], dv)

        def inv(j, _):
            sl = pl.ds(j * 16, 16)
            dv[sl] = 1.0 / (dv[sl] + 1e-16)
            return 0
        lax.fori_loop(0, _STRIPE // 16, inv, 0)
        pltpu.sync_copy(dv, sh_den.at[pl.ds(sbase, _STRIPE)])

        plsc.subcore_barrier()

        def p2(i, _):
            rbase = pl.multiple_of(sub * (_TILE_EDGES // 128) + i * _QROWS, 8)
            la = pltpu.async_copy(t_hbm.at[core, pl.ds(rbase, _QROWS)], tv, s_l)
            lb = pltpu.async_copy(src2_hbm.at[core, pl.ds(rbase, _QROWS)], srcv, s_l)
            lc = pltpu.async_copy(dst_hbm.at[pl.ds(rbase, _QROWS)], dstv, s_t)
            la.wait()
            lb.wait()
            lc.wait()
            gd = [pltpu.async_copy(sh_den.at[dstv.at[q]], gv.at[q], s_g)
                  for q in range(_QROWS)]
            for d in gd:
                d.wait()
            for q in range(_QROWS):
                for j in range(8):
                    sl = pl.ds(j * 16, 16)
                    wv[q, sl] = tv[q, sl] * gv[q, sl]
            for h in range(_QROWS // nb):
                rd = [pltpu.async_copy(tbl_hbm.at[srcv.at[h * nb + b]], rows.at[b], s_r0)
                      for b in range(nb)]
                for d in rd:
                    d.wait()

                def scale_rows(rg, _):
                    for b in range(nb):
                        w16 = wv[h * nb + b, pl.ds(rg * 16, 16)]
                        for k in range(16):
                            r = rg * 16 + k
                            wvec = jnp.broadcast_to(w16[k], (16,))
                            for j in range(ch // 16):
                                sl = pl.ds(j * 16, 16)
                                rows[b, r, sl] = rows[b, r, sl] * wvec
                    return 0
                lax.fori_loop(0, 8, scale_rows, 0)
                sd = [pltpu.async_copy(rows.at[b], sh_acc.at[dstv.at[h * nb + b]],
                                       s_s0, add=True)
                      for b in range(nb)]
                for d in sd:
                    d.wait()
            return 0
        lax.fori_loop(0, _CHUNKS, p2, 0)

        plsc.subcore_barrier()

        def wb(k, _):
            off = pl.multiple_of(sbase + k * 64, 8)
            pltpu.sync_copy(sh_acc.at[pl.ds(off, 64)], zb2)
            pltpu.sync_copy(zb2, out_hbm.at[core, pl.ds(off, 64)])
            return 0
        lax.fori_loop(0, _STRIPE // 64, wb, 0)

    return edge_kernel


_make_edge_kernel = functools.lru_cache(maxsize=None)(_make_edge_kernel)


def _bn_coeffs(sumv, sqv, gamma, beta):
    mean = sumv[0] / _N
    var = sqv[0] / _N - mean * mean
    scale = gamma * lax.rsqrt(var + 1e-5)
    shift = beta - mean * scale
    return scale[None], shift[None]


def kernel(x, edge_index, batch, climber, W1, att_src1, att_dst1, b1, g1, be1,
           W2, att_src2, att_dst2, b2, g2, be2, Wc, bc, Wa, ba, Wb, bb):
    n = x.shape[0]
    loop = jnp.arange(n, dtype=jnp.int32)
    src = jnp.concatenate([edge_index[0].astype(jnp.int32), loop])
    dst = jnp.concatenate([edge_index[1].astype(jnp.int32), loop])
    srcp = jnp.pad(src, (0, _EPAD - _E_TOTAL))
    dstp = jnp.pad(dst, (0, _EPAD - _E_TOTAL))
    src_r = srcp.reshape(_EROWS, 128)
    dst_r = dstp.reshape(_EROWS, 128)
    src2_r = jnp.stack([srcp, srcp + _NPAD]).reshape(2, _EROWS, 128)
    batch_p = jnp.pad(batch.astype(jnp.int32), (0, _NPAD - n)).reshape(_NPAD, 1)

    tbl1, avs1, avd1, ms1, md1 = _enc1(x, W1, att_src1, att_dst1, 2, 32)
    out1, _ = _make_edge_kernel(2, 32)(tbl1.reshape(2 * _NPAD, 32), src_r, src2_r, dst_r,
                                       avs1.reshape(-1), avd1.reshape(-1),
                                       ms1.reshape(-1), md1.reshape(-1))
    sum1, sq1 = _red(out1)
    scale1, shift1 = _bn_coeffs(sum1, sq1, g1, be1)

    tbl2, avs2, avd2, ms2, md2 = _enc2(out1, scale1, shift1, W2, att_src2, att_dst2, 2, 16)
    out2, _ = _make_edge_kernel(1, 16)(tbl2.reshape(2 * _NPAD, 16), src_r, src2_r, dst_r,
                                       avs2.reshape(-1), avd2.reshape(-1),
                                       ms2.reshape(-1), md2.reshape(-1))
    sum2, sq2 = _red(out2)
    scale2, shift2 = _bn_coeffs(sum2, sq2, g2, be2)

    y = _fin(out2, scale2, shift2, batch_p, climber, Wc, bc.reshape(1, -1),
             Wa, ba.reshape(1, -1), Wb, bb.reshape(1, -1))
    return y[:n]


# speedup vs baseline: 81.2151x; 1.0556x over previous
import functools
import jax
import jax.numpy as jnp
from jax import lax
from jax.experimental import pallas as pl
from jax.experimental.pallas import tpu as pltpu
from jax.experimental.pallas import tpu_sc as plsc

_N = 50000
_NPAD = 50176
_STRIPE = _NPAD // 16
_E_TOTAL = 850000
_EPAD = 851968
_EROWS = _EPAD // 128
_ECHUNK = 1024
_TILE_EDGES = _EPAD // 16
_CHUNKS = _TILE_EDGES // _ECHUNK
_QROWS = _ECHUNK // 128
_BLK = 1024
_GRID = _NPAD // _BLK


def _enc_common(h, i, asrc, adst, tbl_ref, avs_ref, avd_ref, ms_ref, md_ref):
    nb = h.shape[0]
    split = tbl_ref.shape[0]
    ch = tbl_ref.shape[2]
    tbl_ref[...] = h.reshape(nb, split, ch).transpose(1, 0, 2)
    heads = avs_ref.shape[0]
    hch = h.shape[1] // heads
    hh = h.reshape(nb, heads, hch)
    avs = (hh * asrc[None]).sum(-1)
    avd = (hh * adst[None]).sum(-1)
    avs_ref[...] = avs.T
    avd_ref[...] = avd.T
    rows = i * _BLK + lax.broadcasted_iota(jnp.int32, (nb, 1), 0)
    valid = rows < _N
    ms = jnp.max(jnp.where(valid, avs, -3.4e38), axis=0)
    md = jnp.max(jnp.where(valid, avd, -3.4e38), axis=0)

    @pl.when(i == 0)
    def _():
        ms_ref[...] = jnp.full(ms_ref.shape, -3.4e38, jnp.float32)
        md_ref[...] = jnp.full(md_ref.shape, -3.4e38, jnp.float32)

    ms_ref[...] = jnp.maximum(ms_ref[...], ms[:, None])
    md_ref[...] = jnp.maximum(md_ref[...], md[:, None])


def _enc1_body(x_ref, w_ref, as_ref, ad_ref, tbl_ref, avs_ref, avd_ref, ms_ref, md_ref):
    i = pl.program_id(0)
    h = jnp.dot(x_ref[...], w_ref[...], preferred_element_type=jnp.float32)
    _enc_common(h, i, as_ref[...], ad_ref[...], tbl_ref, avs_ref, avd_ref, ms_ref, md_ref)


def _enc2_body(sc_ref, scale_ref, shift_ref, w_ref, as_ref, ad_ref,
               tbl_ref, avs_ref, avd_ref, ms_ref, md_ref):
    i = pl.program_id(0)
    t = sc_ref[...]
    chin2 = t.shape[0] * t.shape[2]
    act = t.transpose(1, 0, 2).reshape(_BLK, chin2)
    act = jax.nn.relu(act * scale_ref[...][0][None] + shift_ref[...][0][None])
    h = jnp.dot(act, w_ref[...], preferred_element_type=jnp.float32)
    _enc_common(h, i, as_ref[...], ad_ref[...], tbl_ref, avs_ref, avd_ref, ms_ref, md_ref)


def _enc1(x, w, a_src, a_dst, split, ch):
    heads = a_src.shape[0]
    m = w.shape[1]
    return pl.pallas_call(
        _enc1_body,
        grid=(_GRID,),
        in_specs=[
            pl.BlockSpec((_BLK, x.shape[1]), lambda i: (i, 0)),
            pl.BlockSpec(w.shape, lambda i: (0, 0)),
            pl.BlockSpec(a_src.shape, lambda i: (0, 0)),
            pl.BlockSpec(a_dst.shape, lambda i: (0, 0)),
        ],
        out_specs=[
            pl.BlockSpec((split, _BLK, ch), lambda i: (0, i, 0)),
            pl.BlockSpec((heads, _BLK), lambda i: (0, i)),
            pl.BlockSpec((heads, _BLK), lambda i: (0, i)),
            pl.BlockSpec((heads, 16), lambda i: (0, 0)),
            pl.BlockSpec((heads, 16), lambda i: (0, 0)),
        ],
        out_shape=[
            jax.ShapeDtypeStruct((split, _NPAD, ch), jnp.float32),
            jax.ShapeDtypeStruct((heads, _NPAD), jnp.float32),
            jax.ShapeDtypeStruct((heads, _NPAD), jnp.float32),
            jax.ShapeDtypeStruct((heads, 16), jnp.float32),
            jax.ShapeDtypeStruct((heads, 16), jnp.float32),
        ],
    )(x, w, a_src, a_dst)


def _enc2(sc_in, scale, shift, w, a_src, a_dst, split, ch):
    heads = a_src.shape[0]
    chin = sc_in.shape[2]
    return pl.pallas_call(
        _enc2_body,
        grid=(_GRID,),
        in_specs=[
            pl.BlockSpec((2, _BLK, chin), lambda i: (0, i, 0)),
            pl.BlockSpec(scale.shape, lambda i: (0, 0)),
            pl.BlockSpec(shift.shape, lambda i: (0, 0)),
            pl.BlockSpec(w.shape, lambda i: (0, 0)),
            pl.BlockSpec(a_src.shape, lambda i: (0, 0)),
            pl.BlockSpec(a_dst.shape, lambda i: (0, 0)),
        ],
        out_specs=[
            pl.BlockSpec((split, _BLK, ch), lambda i: (0, i, 0)),
            pl.BlockSpec((heads, _BLK), lambda i: (0, i)),
            pl.BlockSpec((heads, _BLK), lambda i: (0, i)),
            pl.BlockSpec((heads, 16), lambda i: (0, 0)),
            pl.BlockSpec((heads, 16), lambda i: (0, 0)),
        ],
        out_shape=[
            jax.ShapeDtypeStruct((split, _NPAD, ch), jnp.float32),
            jax.ShapeDtypeStruct((heads, _NPAD), jnp.float32),
            jax.ShapeDtypeStruct((heads, _NPAD), jnp.float32),
            jax.ShapeDtypeStruct((heads, 16), jnp.float32),
            jax.ShapeDtypeStruct((heads, 16), jnp.float32),
        ],
    )(sc_in, scale, shift, w, a_src, a_dst)


def _red_body(sc_ref, sum_ref, sq_ref):
    i = pl.program_id(0)
    t = sc_ref[...]
    m = t.transpose(1, 0, 2).reshape(_BLK, t.shape[0] * t.shape[2])

    @pl.when(i == 0)
    def _():
        sum_ref[...] = jnp.zeros(sum_ref.shape, jnp.float32)
        sq_ref[...] = jnp.zeros(sq_ref.shape, jnp.float32)

    sum_ref[...] += m.sum(0)[None]
    sq_ref[...] += (m * m).sum(0)[None]


def _red(sc_in):
    chin = sc_in.shape[2]
    m = 2 * chin
    return pl.pallas_call(
        _red_body,
        grid=(_GRID,),
        in_specs=[pl.BlockSpec((2, _BLK, chin), lambda i: (0, i, 0))],
        out_specs=[
            pl.BlockSpec((1, m), lambda i: (0, 0)),
            pl.BlockSpec((1, m), lambda i: (0, 0)),
        ],
        out_shape=[
            jax.ShapeDtypeStruct((1, m), jnp.float32),
            jax.ShapeDtypeStruct((1, m), jnp.float32),
        ],
    )(sc_in)


def _fin_body(sc_ref, scale_ref, shift_ref, batch_ref, cl_ref, wc_ref, bc_ref,
              wa_ref, ba_ref, wb_ref, bb_ref, out_ref):
    t = sc_ref[...]
    act = t.transpose(1, 0, 2).reshape(_BLK, 32)
    act = jax.nn.relu(act * scale_ref[...][0][None] + shift_ref[...][0][None])
    ce = jax.nn.relu(jnp.dot(cl_ref[...], wc_ref[...], preferred_element_type=jnp.float32)
                     + bc_ref[...][0][None])
    ce2 = jnp.dot(ce, wa_ref[...][32:, :], preferred_element_type=jnp.float32)
    oh = (batch_ref[...] == lax.broadcasted_iota(jnp.int32, (1, 64), 1)).astype(jnp.float32)
    hid = jnp.dot(act, wa_ref[...][:32, :], preferred_element_type=jnp.float32)
    hid = hid + jnp.dot(oh, ce2, preferred_element_type=jnp.float32)
    hid = jax.nn.relu(hid + ba_ref[...][0][None])
    out_ref[...] = jnp.dot(hid, wb_ref[...], preferred_element_type=jnp.float32) + bb_ref[...][0][None]


def _fin(sc_in, scale, shift, batch_p, climber, wc, bc, wa, ba, wb, bb):
    return pl.pallas_call(
        _fin_body,
        grid=(_GRID,),
        in_specs=[
            pl.BlockSpec((2, _BLK, 16), lambda i: (0, i, 0)),
            pl.BlockSpec(scale.shape, lambda i: (0, 0)),
            pl.BlockSpec(shift.shape, lambda i: (0, 0)),
            pl.BlockSpec((_BLK, 1), lambda i: (i, 0)),
            pl.BlockSpec(climber.shape, lambda i: (0, 0)),
            pl.BlockSpec(wc.shape, lambda i: (0, 0)),
            pl.BlockSpec(bc.shape, lambda i: (0, 0)),
            pl.BlockSpec(wa.shape, lambda i: (0, 0)),
            pl.BlockSpec(ba.shape, lambda i: (0, 0)),
            pl.BlockSpec(wb.shape, lambda i: (0, 0)),
            pl.BlockSpec(bb.shape, lambda i: (0, 0)),
        ],
        out_specs=pl.BlockSpec((_BLK, 4), lambda i: (i, 0)),
        out_shape=jax.ShapeDtypeStruct((_NPAD, 4), jnp.float32),
    )(sc_in, scale, shift, batch_p, climber, wc, bc, wa, ba, wb, bb)


def _make_edge_kernel(heads, ch):
    nb = 2 if ch == 32 else 8
    mesh = plsc.VectorSubcoreMesh(core_axis_name="c", subcore_axis_name="s")
    out_type = [
        jax.ShapeDtypeStruct((2, _NPAD, ch), jnp.float32),
        jax.ShapeDtypeStruct((2, _EROWS, 128), jnp.float32),
        jax.ShapeDtypeStruct((2, 2, ch), jnp.float32),
    ]
    scratch = [
        pltpu.VMEM_SHARED((_NPAD,), jnp.float32),
        pltpu.VMEM_SHARED((_NPAD,), jnp.float32),
        pltpu.VMEM_SHARED((_NPAD,), jnp.float32),
        pltpu.VMEM_SHARED((_NPAD, ch), jnp.float32),
        pltpu.VMEM_SHARED((16, ch), jnp.float32),
        pltpu.VMEM((_QROWS, 128), jnp.int32),
        pltpu.VMEM((_QROWS, 128), jnp.int32),
        pltpu.VMEM((_QROWS, 128), jnp.float32),
        pltpu.VMEM((_QROWS, 128), jnp.float32),
        pltpu.VMEM((_QROWS, 128), jnp.float32),
        pltpu.VMEM((_QROWS, 128), jnp.float32),
        pltpu.VMEM((_QROWS, 128), jnp.float32),
        pltpu.VMEM((nb, 128, ch), jnp.float32),
        pltpu.VMEM((_STRIPE,), jnp.float32),
        pltpu.VMEM((64, ch), jnp.float32),
        pltpu.VMEM((16, ch), jnp.float32),
        pltpu.VMEM((16,), jnp.float32),
        pltpu.VMEM((16,), jnp.float32),
        pltpu.SemaphoreType.DMA,
        pltpu.SemaphoreType.DMA,
        pltpu.SemaphoreType.DMA,
        pltpu.SemaphoreType.DMA,
        pltpu.SemaphoreType.DMA,
        pltpu.SemaphoreType.DMA,
        pltpu.SemaphoreType.DMA,
        pltpu.SemaphoreType.DMA,
    ]

    @functools.partial(pl.kernel, out_type=out_type, mesh=mesh, scratch_types=scratch,
                       compiler_params=pltpu.CompilerParams(use_tc_tiling_on_sc=False))
    def edge_kernel(tbl_hbm, src_hbm, src2_hbm, dst_hbm, asrc_hbm, adst_hbm, ms_hbm, md_hbm,
                    out_hbm, t_hbm, st_hbm,
                    sh_asrc, sh_adst, sh_den, sh_acc, sh_st,
                    srcv, dstv, av, bv, tv, gv, wv, rows, dv, zb2, stv, c16a, c16b,
                    s_l, s_t, s_g, s_w, s_r0, s_r1, s_s0, s_s1):
        core = lax.axis_index("c")
        sub = lax.axis_index("s")
        hidx = core if heads == 2 else 0
        sbase = pl.multiple_of(sub * _STRIPE, 8)
        aoff = pl.multiple_of(hidx * _NPAD + sub * _STRIPE, 8)
        moff = pl.multiple_of(hidx * 16, 8)

        pltpu.sync_copy(ms_hbm.at[pl.ds(moff, 16)], c16a)
        pltpu.sync_copy(md_hbm.at[pl.ds(moff, 16)], c16b)
        pltpu.sync_copy(asrc_hbm.at[pl.ds(aoff, _STRIPE)], dv)
        pltpu.sync_copy(dv, sh_asrc.at[pl.ds(sbase, _STRIPE)])
        pltpu.sync_copy(adst_hbm.at[pl.ds(aoff, _STRIPE)], dv)
        pltpu.sync_copy(dv, sh_adst.at[pl.ds(sbase, _STRIPE)])

        def zfill(j, _):
            dv[pl.ds(j * 16, 16)] = jnp.zeros((16,), jnp.float32)
            return 0
        lax.fori_loop(0, _STRIPE // 16, zfill, 0)

        def zfill2(r, _):
            for j in range(ch // 16):
                zb2[r, pl.ds(j * 16, 16)] = jnp.zeros((16,), jnp.float32)
            return 0
        lax.fori_loop(0, 64, zfill2, 0)

        pltpu.sync_copy(dv, sh_den.at[pl.ds(sbase, _STRIPE)])

        def zacc(k, _):
            pltpu.sync_copy(zb2, sh_acc.at[pl.ds(sbase + k * 64, 64)])
            return 0
        lax.fori_loop(0, _STRIPE // 64, zacc, 0)

        for r in range(16):
            for j in range(ch // 16):
                stv[r, pl.ds(j * 16, 16)] = jnp.zeros((16,), jnp.float32)

        @pl.when(sub == 0)
        def _():
            pltpu.sync_copy(stv, sh_st)

        plsc.subcore_barrier()

        cvec = c16a[...] + c16b[...]

        def p1(i, _):
            rbase = pl.multiple_of(sub * (_TILE_EDGES // 128) + i * _QROWS, 8)
            la = pltpu.async_copy(src_hbm.at[pl.ds(rbase, _QROWS)], srcv, s_l)
            lb = pltpu.async_copy(dst_hbm.at[pl.ds(rbase, _QROWS)], dstv, s_l)
            la.wait()
            lb.wait()
            gd = []
            for q in range(_QROWS):
                gd.append(pltpu.async_copy(sh_asrc.at[srcv.at[q]], av.at[q], s_g))
                gd.append(pltpu.async_copy(sh_adst.at[dstv.at[q]], bv.at[q], s_g))
            for d in gd:
                d.wait()

            ebase = rbase * 128
            for q in range(_QROWS):
                for j in range(8):
                    sl = pl.ds(j * 16, 16)
                    e = av[q, sl] + bv[q, sl]
                    e = jnp.maximum(e, 0.2 * e)
                    t = jnp.exp(e - cvec)
                    eid = ebase + q * 128 + j * 16 + lax.broadcasted_iota(jnp.int32, (16,), 0)
                    tv[q, sl] = jnp.where(eid < _E_TOTAL, t, 0.0)
            wd = [pltpu.async_copy(tv.at[q], sh_den.at[dstv.at[q]], s_w, add=True)
                  for q in range(_QROWS)]
            ts = pltpu.async_copy(tv, t_hbm.at[core, pl.ds(rbase, _QROWS)], s_t)
            for d in wd:
                d.wait()
            ts.wait()
            return 0
        lax.fori_loop(0, _CHUNKS, p1, 0)

        plsc.subcore_barrier()

        pltpu.sync_copy(sh_den.at[pl.ds(sbase, _STRIPE)], dv)

        def inv(j, _):
            sl = pl.ds(j * 16, 16)
            dv[sl] = 1.0 / (dv[sl] + 1e-16)
            return 0
        lax.fori_loop(0, _STRIPE // 16, inv, 0)
        pltpu.sync_copy(dv, sh_den.at[pl.ds(sbase, _STRIPE)])

        plsc.subcore_barrier()

        def p2(i, _):
            rbase = pl.multiple_of(sub * (_TILE_EDGES // 128) + i * _QROWS, 8)
            la = pltpu.async_copy(t_hbm.at[core, pl.ds(rbase, _QROWS)], tv, s_l)
            lb = pltpu.async_copy(src2_hbm.at[core, pl.ds(rbase, _QROWS)], srcv, s_l)
            lc = pltpu.async_copy(dst_hbm.at[pl.ds(rbase, _QROWS)], dstv, s_t)
            la.wait()
            lb.wait()
            lc.wait()
            gd = [pltpu.async_copy(sh_den.at[dstv.at[q]], gv.at[q], s_g)
                  for q in range(_QROWS)]
            for d in gd:
                d.wait()
            for q in range(_QROWS):
                for j in range(8):
                    sl = pl.ds(j * 16, 16)
                    wv[q, sl] = tv[q, sl] * gv[q, sl]
            for h in range(_QROWS // nb):
                rd = [pltpu.async_copy(tbl_hbm.at[srcv.at[h * nb + b]], rows.at[b], s_r0)
                      for b in range(nb)]
                for d in rd:
                    d.wait()

                def scale_rows(rg, _):
                    for b in range(nb):
                        w16 = wv[h * nb + b, pl.ds(rg * 16, 16)]
                        for k in range(16):
                            r = rg * 16 + k
                            wvec = jnp.broadcast_to(w16[k], (16,))
                            for j in range(ch // 16):
                                sl = pl.ds(j * 16, 16)
                                rows[b, r, sl] = rows[b, r, sl] * wvec
                    return 0
                lax.fori_loop(0, 8, scale_rows, 0)
                sd = [pltpu.async_copy(rows.at[b], sh_acc.at[dstv.at[h * nb + b]],
                                       s_s0, add=True)
                      for b in range(nb)]
                for d in sd:
                    d.wait()
            return 0
        lax.fori_loop(0, _CHUNKS, p2, 0)

        plsc.subcore_barrier()

        def wb(k, _):
            off = pl.multiple_of(sbase + k * 64, 8)
            pltpu.sync_copy(sh_acc.at[pl.ds(off, 64)], zb2)
            d = pltpu.async_copy(zb2, out_hbm.at[core, pl.ds(off, 64)], s_t)
            for j in range(ch // 16):
                sl = pl.ds(j * 16, 16)
                s0 = stv[0, sl]
                s1 = stv[1, sl]
                for r in range(64):
                    v = zb2[r, sl]
                    s0 = s0 + v
                    s1 = s1 + v * v
                stv[0, sl] = s0
                stv[1, sl] = s1
            d.wait()
            return 0
        lax.fori_loop(0, _STRIPE // 64, wb, 0)
        pltpu.sync_copy(stv, sh_st.at[lax.broadcasted_iota(jnp.int32, (16,), 0)],
                        add=True)

        plsc.subcore_barrier()

        @pl.when(sub == 0)
        def _():
            pltpu.sync_copy(sh_st, stv)
            pltpu.sync_copy(stv.at[pl.ds(0, 2)], st_hbm.at[core])

    return edge_kernel


_make_edge_kernel = functools.lru_cache(maxsize=None)(_make_edge_kernel)


def _bn_coeffs(sumv, sqv, gamma, beta):
    mean = sumv[0] / _N
    var = sqv[0] / _N - mean * mean
    scale = gamma * lax.rsqrt(var + 1e-5)
    shift = beta - mean * scale
    return scale[None], shift[None]


def kernel(x, edge_index, batch, climber, W1, att_src1, att_dst1, b1, g1, be1,
           W2, att_src2, att_dst2, b2, g2, be2, Wc, bc, Wa, ba, Wb, bb):
    n = x.shape[0]
    loop = jnp.arange(n, dtype=jnp.int32)
    src = jnp.concatenate([edge_index[0].astype(jnp.int32), loop])
    dst = jnp.concatenate([edge_index[1].astype(jnp.int32), loop])
    srcp = jnp.pad(src, (0, _EPAD - _E_TOTAL))
    dstp = jnp.pad(dst, (0, _EPAD - _E_TOTAL))
    src_r = srcp.reshape(_EROWS, 128)
    dst_r = dstp.reshape(_EROWS, 128)
    src2_r = jnp.stack([srcp, srcp + _NPAD]).reshape(2, _EROWS, 128)
    batch_p = jnp.pad(batch.astype(jnp.int32), (0, _NPAD - n)).reshape(_NPAD, 1)

    tbl1, avs1, avd1, ms1, md1 = _enc1(x, W1, att_src1, att_dst1, 2, 32)
    out1, _, st1 = _make_edge_kernel(2, 32)(tbl1.reshape(2 * _NPAD, 32), src_r, src2_r,
                                            dst_r, avs1.reshape(-1), avd1.reshape(-1),
                                            ms1.reshape(-1), md1.reshape(-1))
    scale1, shift1 = _bn_coeffs(st1[:, 0, :].reshape(1, -1), st1[:, 1, :].reshape(1, -1),
                                g1, be1)

    tbl2, avs2, avd2, ms2, md2 = _enc2(out1, scale1, shift1, W2, att_src2, att_dst2, 2, 16)
    out2, _, st2 = _make_edge_kernel(1, 16)(tbl2.reshape(2 * _NPAD, 16), src_r, src2_r,
                                            dst_r, avs2.reshape(-1), avd2.reshape(-1),
                                            ms2.reshape(-1), md2.reshape(-1))
    scale2, shift2 = _bn_coeffs(st2[:, 0, :].reshape(1, -1), st2[:, 1, :].reshape(1, -1),
                                g2, be2)

    y = _fin(out2, scale2, shift2, batch_p, climber, Wc, bc.reshape(1, -1),
             Wa, ba.reshape(1, -1), Wb, bb.reshape(1, -1))
    return y[:n]
